# Initial kernel scaffold; baseline (speedup 1.0000x reference)
import jax
import jax.numpy as jnp
from jax.experimental import pallas as pl


def kernel(x, edge_index, edge_type, norm, emb_rel, prev_h, weight_neighbor, loop_weight, evolve_loop_weight):
    raise NotImplementedError("write your pallas kernel here")


# speedup vs baseline: 6.6087x; 6.6087x over previous
import functools

import jax
import jax.numpy as jnp
from jax import lax
from jax.experimental import pallas as pl
from jax.experimental.pallas import tpu as pltpu
from jax.experimental.pallas import tpu_sc as plsc

NC, NS = 2, 16
K = 125
NPAD = 10240
RPT = NPAD // NS
ZROWS = 128


def _sc_segment_sums(xl, xr, src3, dst3, et3, rell, relr):
    n, hd = xl.shape
    r = rell.shape[0]
    nchunk = src3.shape[1]

    mesh = plsc.VectorSubcoreMesh(
        core_axis_name="c", subcore_axis_name="s", num_cores=NC, num_subcores=NS
    )

    @functools.partial(
        pl.kernel,
        out_type=(
            jax.ShapeDtypeStruct((NPAD, hd), jnp.float32),
            jax.ShapeDtypeStruct((NPAD, hd), jnp.float32),
            jax.ShapeDtypeStruct((NPAD,), jnp.float32),
        ),
        mesh=mesh,
        scratch_types=[
            pltpu.VMEM((nchunk, K), jnp.int32),
            pltpu.VMEM((nchunk, K), jnp.int32),
            pltpu.VMEM((nchunk, K), jnp.int32),
            pltpu.VMEM((K, hd), jnp.float32),
            pltpu.VMEM((K, hd), jnp.float32),
            pltpu.VMEM((128,), jnp.float32),
            pltpu.VMEM((ZROWS, hd), jnp.float32),
            pltpu.VMEM((RPT,), jnp.float32),
            pltpu.VMEM_SHARED((NPAD, hd), jnp.float32),
            pltpu.VMEM_SHARED((NPAD,), jnp.float32),
            pltpu.VMEM_SHARED((r, hd), jnp.float32),
            pltpu.SemaphoreType.DMA,
            pltpu.SemaphoreType.DMA,
        ],
        compiler_params=pltpu.CompilerParams(use_tc_tiling_on_sc=False),
    )
    def sc_fn(xl_hbm, xr_hbm, src_hbm, dst_hbm, et_hbm, rell_hbm, relr_hbm,
              s0_hbm, s1_hbm, deg_hbm,
              srcv, dstv, etv, xbuf, rbuf, ones, zbuf, dstage,
              acc, degacc, srel, gsem, ssem):
        c = lax.axis_index("c")
        s = lax.axis_index("s")

        zv = jnp.zeros((16,), jnp.float32)
        ov = jnp.ones((16,), jnp.float32)
        nsub = hd // 16

        def zrow(i, _):
            for j in range(nsub):
                zbuf[i, pl.ds(j * 16, 16)] = zv
            return 0

        lax.fori_loop(0, ZROWS, zrow, 0)

        def zdeg(i, _):
            dstage[pl.ds(i * 16, 16)] = zv
            return 0

        lax.fori_loop(0, RPT // 16, zdeg, 0)
        for j in range(8):
            ones[pl.ds(j * 16, 16)] = ov

        for i in range(RPT // ZROWS):
            pltpu.sync_copy(zbuf, acc.at[pl.ds(s * RPT + i * ZROWS, ZROWS)])
        pltpu.sync_copy(dstage, degacc.at[pl.ds(s * RPT, RPT)])

        @pl.when(jnp.logical_and(s == 0, c == 0))
        def _():
            pltpu.sync_copy(rell_hbm, srel)

        @pl.when(jnp.logical_and(s == 0, c == 1))
        def _():
            pltpu.sync_copy(relr_hbm, srel)

        pltpu.sync_copy(src_hbm.at[s], srcv)
        pltpu.sync_copy(dst_hbm.at[s], dstv)
        pltpu.sync_copy(et_hbm.at[s], etv)

        plsc.subcore_barrier()

        def make_chunk(x_src, with_deg):
            def chunk(j, _):
                pltpu.async_copy(x_src.at[srcv.at[j]], xbuf, gsem).wait()
                pltpu.async_copy(srel.at[etv.at[j]], rbuf, gsem).wait()
                pltpu.async_copy(xbuf, acc.at[dstv.at[j]], ssem, add=True).wait()
                pltpu.async_copy(rbuf, acc.at[dstv.at[j]], ssem, add=True).wait()
                if with_deg:
                    pltpu.async_copy(
                        ones.at[pl.ds(0, K)], degacc.at[dstv.at[j]], ssem,
                        add=True,
                    ).wait()
                return 0

            return chunk

        @pl.when(c == 0)
        def _():
            lax.fori_loop(0, nchunk, make_chunk(xl_hbm, True), 0)

        @pl.when(c == 1)
        def _():
            lax.fori_loop(0, nchunk, make_chunk(xr_hbm, False), 0)

        plsc.subcore_barrier()

        @pl.when(c == 0)
        def _():
            for i in range(RPT // ZROWS):
                rows = pl.ds(s * RPT + i * ZROWS, ZROWS)
                pltpu.sync_copy(acc.at[rows], zbuf)
                pltpu.sync_copy(zbuf, s0_hbm.at[rows])
            pltpu.sync_copy(degacc.at[pl.ds(s * RPT, RPT)], dstage)
            pltpu.sync_copy(dstage, deg_hbm.at[pl.ds(s * RPT, RPT)])

        @pl.when(c == 1)
        def _():
            for i in range(RPT // ZROWS):
                rows = pl.ds(s * RPT + i * ZROWS, ZROWS)
                pltpu.sync_copy(acc.at[rows], zbuf)
                pltpu.sync_copy(zbuf, s1_hbm.at[rows])

    return sc_fn(xl, xr, src3, dst3, et3, rell, relr)


def _tc_combine(s0, s1, x, norm, deg, wn, wl, we):
    n, d = x.shape
    hd = d // 2
    bs = 512

    def body(s0_ref, s1_ref, x_ref, norm_ref, deg_ref,
             wn_ref, wl_ref, we_ref, o_ref):
        h = jnp.dot(s0_ref[...], wn_ref[0:hd, :],
                    preferred_element_type=jnp.float32)
        h = h + jnp.dot(s1_ref[...], wn_ref[hd:d, :],
                        preferred_element_type=jnp.float32)
        xb = x_ref[...]
        lm_main = jnp.dot(xb, wl_ref[...], preferred_element_type=jnp.float32)
        lm_evo = jnp.dot(xb, we_ref[...], preferred_element_type=jnp.float32)
        o_ref[...] = h * norm_ref[...] + jnp.where(
            deg_ref[...] > 0.0, lm_main, lm_evo)

    half_spec = pl.BlockSpec((bs, hd), lambda i: (i, 0))
    row_spec = pl.BlockSpec((bs, d), lambda i: (i, 0))
    col_spec = pl.BlockSpec((bs, 1), lambda i: (i, 0))
    w_spec = pl.BlockSpec((d, d), lambda i: (0, 0))

    return pl.pallas_call(
        body,
        grid=(NPAD // bs,),
        in_specs=[half_spec, half_spec, row_spec, col_spec, col_spec,
                  w_spec, w_spec, w_spec],
        out_specs=row_spec,
        out_shape=jax.ShapeDtypeStruct((n, d), jnp.float32),
    )(s0, s1, x, norm, deg, wn, wl, we)


def kernel(x, edge_index, edge_type, norm, emb_rel, prev_h,
           weight_neighbor, loop_weight, evolve_loop_weight):
    n, d = x.shape
    e = edge_type.shape[0]
    hd = d // 2
    nchunk = e // (NS * K)
    assert e == NS * K * nchunk and n <= NPAD

    src3 = edge_index[0].reshape(NS, nchunk, K)
    dst3 = edge_index[1].reshape(NS, nchunk, K)
    et3 = edge_type.reshape(NS, nchunk, K)
    xl = x[:, :hd]
    xr = x[:, hd:]
    rell = emb_rel[:, :hd]
    relr = emb_rel[:, hd:]

    s0, s1, deg = _sc_segment_sums(xl, xr, src3, dst3, et3, rell, relr)
    return _tc_combine(s0, s1, x, norm, deg.reshape(NPAD, 1),
                       weight_neighbor, loop_weight, evolve_loop_weight)


# speedup vs baseline: 7.1730x; 1.0854x over previous
import functools

import jax
import jax.numpy as jnp
from jax import lax
from jax.experimental import pallas as pl
from jax.experimental.pallas import tpu as pltpu
from jax.experimental.pallas import tpu_sc as plsc

NC, NS = 2, 16
K = 125
NQ = 4
NPAD = 10240
RPT = NPAD // NS
ZROWS = 128


def _sc_segment_sums(xl, xr, src3, dst3, et3, rell, relr):
    n, hd = xl.shape
    r = rell.shape[0]
    nchunk = src3.shape[1]

    mesh = plsc.VectorSubcoreMesh(
        core_axis_name="c", subcore_axis_name="s", num_cores=NC, num_subcores=NS
    )

    @functools.partial(
        pl.kernel,
        out_type=(
            jax.ShapeDtypeStruct((NPAD, hd), jnp.float32),
            jax.ShapeDtypeStruct((NPAD, hd), jnp.float32),
            jax.ShapeDtypeStruct((NPAD,), jnp.float32),
        ),
        mesh=mesh,
        scratch_types=[
            pltpu.VMEM((nchunk // NQ, K), jnp.int32),
            pltpu.VMEM((nchunk // NQ, K), jnp.int32),
            pltpu.VMEM((nchunk // NQ, K), jnp.int32),
            pltpu.VMEM((K, hd), jnp.float32),
            pltpu.VMEM((K, hd), jnp.float32),
            pltpu.VMEM((K, hd), jnp.float32),
            pltpu.VMEM((K, hd), jnp.float32),
            pltpu.VMEM((128,), jnp.float32),
            pltpu.VMEM((ZROWS, hd), jnp.float32),
            pltpu.VMEM((RPT,), jnp.float32),
            pltpu.VMEM_SHARED((NPAD, hd), jnp.float32),
            pltpu.VMEM_SHARED((NPAD,), jnp.float32),
            pltpu.VMEM_SHARED((r, hd), jnp.float32),
            pltpu.SemaphoreType.DMA,
            pltpu.SemaphoreType.DMA,
            pltpu.SemaphoreType.DMA,
            pltpu.SemaphoreType.DMA,
        ],
        compiler_params=pltpu.CompilerParams(use_tc_tiling_on_sc=False),
    )
    def sc_fn(xl_hbm, xr_hbm, src_hbm, dst_hbm, et_hbm, rell_hbm, relr_hbm,
              s0_hbm, s1_hbm, deg_hbm,
              srcv, dstv, etv, xb0, xb1, rb0, rb1, ones, zbuf, dstage,
              acc, degacc, srel, gsem0, gsem1, ssem0, ssem1):
        c = lax.axis_index("c")
        s = lax.axis_index("s")

        zv = jnp.zeros((16,), jnp.float32)
        ov = jnp.ones((16,), jnp.float32)
        nsub = hd // 16

        def zrow(i, _):
            for j in range(nsub):
                zbuf[i, pl.ds(j * 16, 16)] = zv
            return 0

        lax.fori_loop(0, ZROWS, zrow, 0)

        def zdeg(i, _):
            dstage[pl.ds(i * 16, 16)] = zv
            return 0

        lax.fori_loop(0, RPT // 16, zdeg, 0)
        for j in range(8):
            ones[pl.ds(j * 16, 16)] = ov

        for i in range(RPT // ZROWS):
            pltpu.sync_copy(zbuf, acc.at[pl.ds(s * RPT + i * ZROWS, ZROWS)])
        pltpu.sync_copy(dstage, degacc.at[pl.ds(s * RPT, RPT)])

        @pl.when(jnp.logical_and(s == 0, c == 0))
        def _():
            pltpu.sync_copy(rell_hbm, srel)

        @pl.when(jnp.logical_and(s == 0, c == 1))
        def _():
            pltpu.sync_copy(relr_hbm, srel)

        plsc.subcore_barrier()

        def make_loop(x_src, with_deg):
            qchunk = nchunk // NQ
            for q in range(NQ):
                qs = pl.ds(q * qchunk, qchunk)
                pltpu.sync_copy(src_hbm.at[s, qs], srcv)
                pltpu.sync_copy(dst_hbm.at[s, qs], dstv)
                pltpu.sync_copy(et_hbm.at[s, qs], etv)

                def pair(jj, _):
                    j0 = 2 * jj
                    j1 = j0 + 1
                    gx0 = pltpu.async_copy(x_src.at[srcv.at[j0]], xb0, gsem0)
                    gr0 = pltpu.async_copy(srel.at[etv.at[j0]], rb0, gsem1)
                    gx0.wait()
                    gr0.wait()
                    pltpu.async_copy(xb0, acc.at[dstv.at[j0]], ssem0,
                                     add=True).wait()
                    pltpu.async_copy(rb0, acc.at[dstv.at[j0]], ssem0,
                                     add=True).wait()
                    if with_deg:
                        pltpu.async_copy(ones.at[pl.ds(0, K)],
                                         degacc.at[dstv.at[j0]], ssem0,
                                         add=True).wait()
                    gx1 = pltpu.async_copy(x_src.at[srcv.at[j1]], xb1, gsem0)
                    gr1 = pltpu.async_copy(srel.at[etv.at[j1]], rb1, gsem1)
                    gx1.wait()
                    gr1.wait()
                    pltpu.async_copy(xb1, acc.at[dstv.at[j1]], ssem1,
                                     add=True).wait()
                    pltpu.async_copy(rb1, acc.at[dstv.at[j1]], ssem1,
                                     add=True).wait()
                    return 0

                lax.fori_loop(0, qchunk // 2, pair, 0)

        @pl.when(c == 0)
        def _():
            make_loop(xl_hbm, True)

        @pl.when(c == 1)
        def _():
            make_loop(xr_hbm, False)

        plsc.subcore_barrier()

        @pl.when(c == 0)
        def _():
            for i in range(RPT // ZROWS):
                rows = pl.ds(s * RPT + i * ZROWS, ZROWS)
                pltpu.sync_copy(acc.at[rows], zbuf)
                pltpu.sync_copy(zbuf, s0_hbm.at[rows])
            pltpu.sync_copy(degacc.at[pl.ds(s * RPT, RPT)], dstage)
            pltpu.sync_copy(dstage, deg_hbm.at[pl.ds(s * RPT, RPT)])

        @pl.when(c == 1)
        def _():
            for i in range(RPT // ZROWS):
                rows = pl.ds(s * RPT + i * ZROWS, ZROWS)
                pltpu.sync_copy(acc.at[rows], zbuf)
                pltpu.sync_copy(zbuf, s1_hbm.at[rows])

    return sc_fn(xl, xr, src3, dst3, et3, rell, relr)


def _tc_combine(s0, s1, x, norm, deg, wn, wl, we):
    n, d = x.shape
    hd = d // 2
    bs = 512

    def body(s0_ref, s1_ref, x_ref, norm_ref, deg_ref,
             wn_ref, wl_ref, we_ref, o_ref):
        h = jnp.dot(s0_ref[...], wn_ref[0:hd, :],
                    preferred_element_type=jnp.float32)
        h = h + jnp.dot(s1_ref[...], wn_ref[hd:d, :],
                        preferred_element_type=jnp.float32)
        xb = x_ref[...]
        lm_main = jnp.dot(xb, wl_ref[...], preferred_element_type=jnp.float32)
        lm_evo = jnp.dot(xb, we_ref[...], preferred_element_type=jnp.float32)
        o_ref[...] = h * norm_ref[...] + jnp.where(
            deg_ref[...] > 0.0, lm_main, lm_evo)

    half_spec = pl.BlockSpec((bs, hd), lambda i: (i, 0))
    row_spec = pl.BlockSpec((bs, d), lambda i: (i, 0))
    col_spec = pl.BlockSpec((bs, 1), lambda i: (i, 0))
    w_spec = pl.BlockSpec((d, d), lambda i: (0, 0))

    return pl.pallas_call(
        body,
        grid=(NPAD // bs,),
        in_specs=[half_spec, half_spec, row_spec, col_spec, col_spec,
                  w_spec, w_spec, w_spec],
        out_specs=row_spec,
        out_shape=jax.ShapeDtypeStruct((n, d), jnp.float32),
    )(s0, s1, x, norm, deg, wn, wl, we)


def kernel(x, edge_index, edge_type, norm, emb_rel, prev_h,
           weight_neighbor, loop_weight, evolve_loop_weight):
    n, d = x.shape
    e = edge_type.shape[0]
    hd = d // 2
    nchunk = e // (NS * K)
    assert e == NS * K * nchunk and n <= NPAD

    src3 = edge_index[0].reshape(NS, nchunk, K)
    dst3 = edge_index[1].reshape(NS, nchunk, K)
    et3 = edge_type.reshape(NS, nchunk, K)
    xl = x[:, :hd]
    xr = x[:, hd:]
    rell = emb_rel[:, :hd]
    relr = emb_rel[:, hd:]

    s0, s1, deg = _sc_segment_sums(xl, xr, src3, dst3, et3, rell, relr)
    return _tc_combine(s0, s1, x, norm, deg.reshape(NPAD, 1),
                       weight_neighbor, loop_weight, evolve_loop_weight)
